# Initial kernel scaffold; baseline (speedup 1.0000x reference)
import jax
import jax.numpy as jnp
from jax.experimental import pallas as pl


def kernel(input_ids, image_features, W):
    raise NotImplementedError("write your pallas kernel here")


# speedup vs baseline: 2.0919x; 2.0919x over previous
import functools

import jax
import jax.numpy as jnp
from jax import lax
from jax.experimental import pallas as pl
from jax.experimental.pallas import tpu as pltpu
from jax.experimental.pallas import tpu_sc as plsc

_VOCAB = 262208
_D = 768
_B = 4
_S = 2048
_IMG_TOKENS = 256
_IMAGE_TOKEN_INDEX = 262144

_N = _B * _S
_NIMG = _B * _IMG_TOKENS
_NC = 2
_NS = 16
_NW = _NC * _NS
_CHUNK = _N // _NW
_L = 16
_VPC = _CHUNK // _L
_SUB = 64
_NSUB = _CHUNK // _SUB

_mesh = plsc.VectorSubcoreMesh(core_axis_name="c", subcore_axis_name="s")


@functools.partial(
    pl.kernel,
    mesh=_mesh,
    out_type=jax.ShapeDtypeStruct((_N, _D), jnp.float32),
    compiler_params=pltpu.CompilerParams(needs_layout_passes=False),
    scratch_types=[
        pltpu.VMEM((_N,), jnp.int32),
        pltpu.VMEM((_CHUNK + _L,), jnp.int32),
        pltpu.VMEM((_CHUNK + _L,), jnp.int32),
        pltpu.VMEM((_SUB, _D), jnp.float32),
        pltpu.VMEM((_SUB, _D), jnp.float32),
        pltpu.SemaphoreType.DMA,
        pltpu.SemaphoreType.DMA,
    ],
)
def _sc_embed(ids_hbm, feats_hbm, w_hbm, out_hbm,
              ids_v, rloc_v, rimg_v, buf0, buf1, sem0, sem1):
    wid = lax.axis_index("s") * _NC + lax.axis_index("c")
    base = wid * _CHUNK
    bufs = (buf0, buf1)
    sems = (sem0, sem1)

    pltpu.sync_copy(ids_hbm, ids_v)

    copies = [None] * _NSUB
    for sub in range(min(2, _NSUB)):
        idx = ids_v.at[pl.ds(base + sub * _SUB, _SUB)]
        copies[sub] = pltpu.async_copy(w_hbm.at[idx], bufs[sub % 2],
                                       sems[sub % 2])

    def _count(i, acc):
        v = ids_v[pl.ds(i * _L, _L)]
        return acc + (v == _IMAGE_TOKEN_INDEX).astype(jnp.int32)

    accv = lax.fori_loop(0, wid * _VPC, _count, jnp.zeros((_L,), jnp.int32))
    offset = jnp.sum(accv)

    lane = lax.iota(jnp.int32, _L)
    cnt = jnp.int32(0)
    bounds = [jnp.int32(0)]
    for j in range(_VPC):
        v = ids_v[pl.ds(base + j * _L, _L)]
        m = v == _IMAGE_TOKEN_INDEX
        mi = m.astype(jnp.int32)
        c = plsc.cumsum(mi)
        pos = offset + cnt + c - 1
        img_idx = jnp.minimum(pos, _NIMG - 1)
        rowloc = j * _L + lane
        plsc.store_compressed(rloc_v.at[pl.ds(cnt, _L)], rowloc, mask=m)
        plsc.store_compressed(rimg_v.at[pl.ds(cnt, _L)], img_idx, mask=m)
        cnt = cnt + c[_L - 1]
        if j % (_SUB // _L) == (_SUB // _L) - 1:
            bounds.append(cnt)

    for sub in range(_NSUB):
        buf = bufs[sub % 2]
        copies[sub].wait()

        def _fix(e, _, sub=sub, buf=buf):
            rl = rloc_v[pl.ds(e, _L)][0]
            ri = rimg_v[pl.ds(e, _L)][0]
            pltpu.sync_copy(feats_hbm.at[pl.ds(ri, 1)],
                            buf.at[pl.ds(rl - sub * _SUB, 1)])
            return 0

        lax.fori_loop(bounds[sub], bounds[sub + 1], _fix, 0)
        pltpu.sync_copy(buf, out_hbm.at[pl.ds(base + sub * _SUB, _SUB)])
        if sub + 2 < _NSUB:
            idx = ids_v.at[pl.ds(base + (sub + 2) * _SUB, _SUB)]
            copies[sub + 2] = pltpu.async_copy(w_hbm.at[idx], buf,
                                               sems[sub % 2])


def kernel(input_ids, image_features, W):
    ids = input_ids.reshape(_N)
    feats = image_features.reshape(_NIMG, _D)
    out = _sc_embed(ids, feats, W)
    return out.reshape(_B, _S, _D)
